# baseline (speedup 1.0000x reference)
import functools

import jax
import jax.numpy as jnp
from jax import lax
from jax.experimental import pallas as pl
from jax.experimental.pallas import tpu as pltpu
from jax.experimental.pallas import tpu_sc as plsc

VOCAB = 100000
N_EMBD = 768
N_POS = 2048
BATCH = 4
SEQ = 2048

NTOK = BATCH * SEQ
NW = 32
ROWS_PER_W = SEQ // NW
PGRP = 16
NGRP = ROWS_PER_W // PGRP
LANES = 16
VECS_PER_ROW = N_EMBD // LANES
NIDX = BATCH * ROWS_PER_W


def _emb_body(ids_hbm, wte_hbm, wpe_hbm, out_hbm,
              idx_v, pos0, pos1,
              t0, t1, t2, t3, t4, t5, t6, t7,
              gs0, gs1, ss0, ss1, ps0, ps1):
    core = lax.axis_index("c")
    sub = lax.axis_index("s")
    wid = sub * 2 + core
    pos_base = wid * ROWS_PER_W

    toks = [t0, t1, t2, t3, t4, t5, t6, t7]
    posb = [pos0, pos1]
    gsems = [gs0, gs1]
    ssems = [ss0, ss1]
    psems = [ps0, ps1]

    for b in range(BATCH):
        pltpu.sync_copy(ids_hbm.at[pl.ds(b * SEQ + pos_base, ROWS_PER_W)],
                        idx_v.at[pl.ds(b * ROWS_PER_W, ROWS_PER_W)])

    def issue_pos(h):
        return pltpu.async_copy(
            wpe_hbm.at[pl.ds(pos_base + h * PGRP, PGRP)],
            posb[h % 2], psems[h % 2])

    def issue_gathers(h):
        return [
            pltpu.async_copy(
                wte_hbm.at[idx_v.at[pl.ds(b * ROWS_PER_W + h * PGRP, PGRP)]],
                toks[(h % 2) * BATCH + b], gsems[h % 2])
            for b in range(BATCH)
        ]

    def issue_stores(h):
        return [
            pltpu.async_copy(
                toks[(h % 2) * BATCH + b],
                out_hbm.at[pl.ds(b * SEQ + pos_base + h * PGRP, PGRP)],
                ssems[h % 2])
            for b in range(BATCH)
        ]

    pos_hs = {0: issue_pos(0)}
    gaths = {0: issue_gathers(0)}
    stores = {}
    for h in range(NGRP):
        if h + 1 < NGRP:
            if h - 1 >= 0:
                for s in stores.pop(h - 1):
                    s.wait()
            gaths[h + 1] = issue_gathers(h + 1)
            pos_hs[h + 1] = issue_pos(h + 1)
        for g in gaths.pop(h):
            g.wait()
        pos_hs.pop(h).wait()

        quad = toks[(h % 2) * BATCH:(h % 2) * BATCH + BATCH]
        pv = posb[h % 2]

        @plsc.parallel_loop(0, PGRP)
        def add_pos(p, quad=quad, pv=pv):
            for k in range(VECS_PER_ROW):
                sl = pl.ds(k * LANES, LANES)
                pos_vec = pv[p, sl]
                for b in range(BATCH):
                    plsc.addupdate(quad[b].at[p, sl], pos_vec)

        stores[h] = issue_stores(h)

    for h in sorted(stores):
        for s in stores[h]:
            s.wait()


_emb = functools.partial(
    pl.kernel,
    mesh=plsc.VectorSubcoreMesh(core_axis_name="c", subcore_axis_name="s"),
    out_type=jax.ShapeDtypeStruct((NTOK, N_EMBD), jnp.float32),
    scratch_types=(
        [pltpu.VMEM((NIDX,), jnp.int32)]
        + [pltpu.VMEM((PGRP, N_EMBD), jnp.float32)] * 2
        + [pltpu.VMEM((PGRP, N_EMBD), jnp.float32)] * 8
        + [pltpu.SemaphoreType.DMA] * 6
    ),
)(_emb_body)


@jax.jit
def kernel(input_ids, wte, wpe):
    ids_flat = input_ids.reshape(-1).astype(jnp.int32)
    out = _emb(ids_flat, wte, wpe)
    return out.reshape(BATCH, SEQ, N_EMBD)
